# baseline (speedup 1.0000x reference)
import functools
import jax
import jax.numpy as jnp
from jax import lax
from jax.experimental import pallas as pl
from jax.experimental.pallas import tpu as pltpu
from jax.experimental.pallas import tpu_sc as plsc

_HUMAN_IDX = 0
_SCORE_TH = 0.2
_IOU_TH = 0.5
_MAX_INST = 15
_N = 5000
_NP = 5120
_R = 1024
_NT = _NP // _R

_NW = 16
_SL = _NP // _NW
_SLC = _SL // 16
_BIG = 1e9
_NEG = float("-inf")


def _nms_kernel(cols_ref, rows_ref, keep_ref, adj_ref):
    f32 = jnp.float32

    x0j = rows_ref[0:1, :]
    y0j = rows_ref[1:2, :]
    x1j = rows_ref[2:3, :]
    y1j = rows_ref[3:4, :]
    sj = rows_ref[4:5, :]
    labj = rows_ref[5:6, :]

    max_coord = jnp.max(cols_ref[:, 0:4])
    off_j = labj * (max_coord + 1.0)
    ox0j = x0j + off_j
    oy0j = y0j + off_j
    ox1j = x1j + off_j
    oy1j = y1j + off_j
    area_jp = (ox1j - ox0j) * (oy1j - oy0j) + 1e-9
    jidx = lax.broadcasted_iota(jnp.int32, (1, _NP), 1).astype(f32)

    keep_ref[...] = jnp.ones((_NP, 1), f32)

    def apply_tile(t, adj, changes):
        rows = pl.ds(t * _R, _R)
        supp = lax.dot_general(
            adj, keep_ref[...],
            (((1,), (0,)), ((), ())),
            preferred_element_type=f32)
        new = jnp.where(supp > 0.5, 0.0, 1.0)
        old = keep_ref[rows]
        keep_ref[rows] = new
        return changes + jnp.sum(jnp.abs(new - old))

    def tile_first(t, changes):
        rows = pl.ds(t * _R, _R)
        off_i = cols_ref[rows, 5:6] * (max_coord + 1.0)
        x0i = cols_ref[rows, 0:1] + off_i
        y0i = cols_ref[rows, 1:2] + off_i
        x1i = cols_ref[rows, 2:3] + off_i
        y1i = cols_ref[rows, 3:4] + off_i
        ai = (x1i - x0i) * (y1i - y0i)
        si = cols_ref[rows, 4:5]
        ii = (lax.broadcasted_iota(jnp.int32, (_R, 1), 0)
              + t * _R).astype(f32)

        wx = jnp.maximum(jnp.minimum(x1i, ox1j) - jnp.maximum(x0i, ox0j), 0.0)
        wy = jnp.maximum(jnp.minimum(y1i, oy1j) - jnp.maximum(y0i, oy0j), 0.0)
        inter = wx * wy
        over = inter * 3.0 > ai + area_jp
        precede = (sj > si) | ((sj == si) & (jidx < ii))
        adj = (over & precede).astype(f32)
        adj_ref[rows, :] = adj.astype(jnp.int8)
        return apply_tile(t, adj, changes)

    def tile_cached(t, changes):
        adj = adj_ref[pl.ds(t * _R, _R), :].astype(f32)
        return apply_tile(t, adj, changes)

    def sweep_body(c):
        return lax.fori_loop(0, _NT, tile_cached, 0.0)

    c1 = lax.fori_loop(0, _NT, tile_first, 0.0)
    lax.while_loop(lambda c: c > 0.0, sweep_body, c1)


def _perm(v, idx16):
    return lax.gather(
        v, idx16[:, None],
        lax.GatherDimensionNumbers(
            offset_dims=(), collapsed_slice_dims=(0,), start_index_map=(0,)),
        (1,), mode=lax.GatherScatterMode.PROMISE_IN_BOUNDS)


def _xmax(v):
    lanei = lax.iota(jnp.int32, 16)
    for k in (1, 2, 4, 8):
        v = jnp.maximum(v, _perm(v, lanei ^ k))
    return v


def _xmin(v):
    lanei = lax.iota(jnp.int32, 16)
    for k in (1, 2, 4, 8):
        v = jnp.minimum(v, _perm(v, lanei ^ k))
    return v


def _topk_rounds(read_chunk, nchunks):
    f32 = jnp.float32
    lane = lax.iota(jnp.int32, 16).astype(f32)

    def round_body(r, carry):
        lm, li, resv, resi = carry

        def p1(k, macc):
            v, gi = read_chunk(k)
            elig = (v < lm) | ((v == lm) & (gi > li))
            return jnp.maximum(macc, jnp.where(elig, v, _NEG))

        macc = lax.fori_loop(0, nchunks, p1, jnp.full((16,), _NEG, f32))
        m = _xmax(macc)

        def p2(k, iacc):
            v, gi = read_chunk(k)
            elig = (v < lm) | ((v == lm) & (gi > li))
            hit = elig & (v == m)
            return jnp.minimum(iacc, jnp.where(hit, gi, _BIG))

        iacc = lax.fori_loop(0, nchunks, p2, jnp.full((16,), _BIG, f32))
        idx = _xmin(iacc)
        rf = r.astype(f32)
        resv = jnp.where(lane == rf, m, resv)
        resi = jnp.where(lane == rf, idx, resi)
        return m, idx, resv, resi

    init = (jnp.full((16,), float("inf"), f32), jnp.full((16,), -1.0, f32),
            jnp.full((16,), _NEG, f32), jnp.full((16,), _BIG, f32))
    _, _, resv, resi = lax.fori_loop(0, _MAX_INST, round_body, init)
    return resv, resi


def _select_body(scores_hbm, labels_hbm, keep_hbm, brow_hbm, hid_hbm,
                 hs_out, bx_out,
                 sv, lv, kv, humv, objv, res_stage, shared, merged,
                 idxv, hsv, bxg, sem):
    f32 = jnp.float32
    cid = lax.axis_index("c")
    sid = lax.axis_index("s")
    lane = lax.iota(jnp.int32, 16).astype(f32)

    @pl.when(cid == 0)
    def _local():
        base = sid * _SL
        pltpu.sync_copy(scores_hbm.at[pl.ds(base, _SL)], sv)
        pltpu.sync_copy(labels_hbm.at[pl.ds(base, _SL)], lv)
        pltpu.sync_copy(keep_hbm.at[pl.ds(base, _SL)], kv)

        def mk_masks(k, _):
            sl = pl.ds(k * 16, 16)
            v = sv[sl]
            l = lv[sl]
            kp = kv[sl]
            valid = (kp > 0.5) & (v >= _SCORE_TH)
            humv[sl] = jnp.where(valid & (l == float(_HUMAN_IDX)), v, _NEG)
            objv[sl] = jnp.where(valid & (l != float(_HUMAN_IDX)), v, _NEG)
            return 0

        lax.fori_loop(0, _SLC, mk_masks, 0)
        basef = (sid * _SL).astype(f32)

        def rd_hum(k):
            gi = basef + (k * 16).astype(f32) + lane
            return humv[pl.ds(k * 16, 16)], gi

        def rd_obj(k):
            gi = basef + (k * 16).astype(f32) + lane
            return objv[pl.ds(k * 16, 16)], gi

        hv, hi = _topk_rounds(rd_hum, _SLC)
        ov, oi = _topk_rounds(rd_obj, _SLC)
        res_stage[pl.ds(0, 16)] = hv
        res_stage[pl.ds(16, 16)] = hi
        res_stage[pl.ds(32, 16)] = ov
        res_stage[pl.ds(48, 16)] = oi
        pltpu.sync_copy(res_stage, shared.at[pl.ds(sid * 64, 64)])

    plsc.subcore_barrier()

    @pl.when((cid == 0) & (sid == 0))
    def _merge_and_gather():
        pltpu.sync_copy(shared, merged)

        def rd_cat(off):
            def rd(k):
                return (merged[pl.ds(k * 64 + off, 16)],
                        merged[pl.ds(k * 64 + off + 16, 16)])
            return rd

        ghv, ghi = _topk_rounds(rd_cat(0), _NW)
        gov, goi = _topk_rounds(rd_cat(32), _NW)

        hum_i = jnp.minimum(ghi, _BIG - 1.0).astype(jnp.int32)
        obj_i = jnp.minimum(goi, _BIG - 1.0).astype(jnp.int32)
        hum_i = jnp.where(hum_i >= _NP, 0, hum_i)
        obj_i = jnp.where(obj_i >= _NP, 0, obj_i)
        idxv[pl.ds(0, 16)] = hum_i
        idxv[pl.ds(16, 16)] = obj_i

        pltpu.async_copy(hid_hbm.at[idxv], hsv, sem).wait()
        pltpu.async_copy(brow_hbm.at[idxv], bxg, sem).wait()

        four = jnp.full((16,), 4, jnp.int32)

        def mul_rows(r, _):
            rowh = bxg[r, pl.ds(0, 16)]
            rowo = bxg[r + 16, pl.ds(0, 16)]
            s_h = _perm(rowh, four)
            s_o = _perm(rowo, four)
            sig_h = 1.0 / (1.0 + jnp.exp(-s_h))
            sig_o = 1.0 / (1.0 + jnp.exp(-s_o))

            def mul_chunk(c, _c):
                hsv[r, pl.ds(c * 16, 16)] = hsv[r, pl.ds(c * 16, 16)] * sig_h
                ro = r + 16
                hsv[ro, pl.ds(c * 16, 16)] = hsv[ro, pl.ds(c * 16, 16)] * sig_o
                return 0

            lax.fori_loop(0, 16, mul_chunk, 0)
            return 0

        lax.fori_loop(0, 16, mul_rows, 0)

        pltpu.sync_copy(hsv, hs_out)
        pltpu.sync_copy(bxg, bx_out)


def _make_select():
    mesh = plsc.VectorSubcoreMesh(core_axis_name="c", subcore_axis_name="s")
    f32 = jnp.float32
    return pl.kernel(
        _select_body,
        mesh=mesh,
        out_type=[
            jax.ShapeDtypeStruct((32, 256), f32),
            jax.ShapeDtypeStruct((32, 128), f32),
        ],
        scratch_types=[
            pltpu.VMEM((_SL,), f32),
            pltpu.VMEM((_SL,), f32),
            pltpu.VMEM((_SL,), f32),
            pltpu.VMEM((_SL,), f32),
            pltpu.VMEM((_SL,), f32),
            pltpu.VMEM((64,), f32),
            pltpu.VMEM_SHARED((_NW * 64,), f32),
            pltpu.VMEM((_NW * 64,), f32),
            pltpu.VMEM((32,), jnp.int32),
            pltpu.VMEM((32, 256), f32),
            pltpu.VMEM((32, 128), f32),
            pltpu.SemaphoreType.DMA,
        ],
    )


def kernel(boxes, scores, hidden_states, labels):
    f32 = jnp.float32
    pad = _NP - _N
    boxesp = jnp.pad(boxes, ((0, pad), (0, 0)))
    scoresp = jnp.pad(scores, (0, pad), constant_values=-jnp.inf)
    labelsp = jnp.pad(labels.astype(f32), (0, pad))
    zeros2 = jnp.zeros((_NP, 2), f32)
    cols = jnp.concatenate(
        [boxesp, scoresp[:, None], labelsp[:, None], zeros2], axis=1)
    rows = cols.T

    keep = pl.pallas_call(
        _nms_kernel,
        out_shape=jax.ShapeDtypeStruct((_NP, 1), f32),
        scratch_shapes=[
            pltpu.VMEM((_NP, _NP), jnp.int8),
        ],
    )(cols, rows)

    brow = jnp.pad(cols, ((0, 0), (0, 120)))
    hs32, bxg = _make_select()(
        scoresp, labelsp, keep.reshape(_NP), brow, hidden_states)
    hs = jnp.concatenate([hs32[0:_MAX_INST], hs32[16:16 + _MAX_INST]], axis=0)
    bx32 = bxg[:, 0:4]
    bx = jnp.concatenate([bx32[0:_MAX_INST], bx32[16:16 + _MAX_INST]], axis=0)
    return jnp.concatenate([hs, bx], axis=1)


# speedup vs baseline: 1.0762x; 1.0762x over previous
import functools
import jax
import jax.numpy as jnp
from jax import lax
from jax.experimental import pallas as pl
from jax.experimental.pallas import tpu as pltpu
from jax.experimental.pallas import tpu_sc as plsc

_HUMAN_IDX = 0
_SCORE_TH = 0.2
_IOU_TH = 0.5
_MAX_INST = 15
_N = 5000
_NP = 5120
_R = 512
_NT = _NP // _R

_NW = 16
_SL = _NP // _NW
_SLC = _SL // 16
_BIG = 1e9
_NEG = float("-inf")


def _nms_kernel(cols_ref, rows_ref, keep_out, keep_ref, adj_ref):
    f32 = jnp.float32
    i8 = jnp.int8

    x0j = rows_ref[0:1, :]
    y0j = rows_ref[1:2, :]
    x1j = rows_ref[2:3, :]
    y1j = rows_ref[3:4, :]
    sj = rows_ref[4:5, :]
    labj = rows_ref[5:6, :]

    max_coord = jnp.max(cols_ref[:, 0:4])
    off_j = labj * (max_coord + 1.0)
    ox0j = x0j + off_j
    oy0j = y0j + off_j
    ox1j = x1j + off_j
    oy1j = y1j + off_j
    area_jp = (ox1j - ox0j) * (oy1j - oy0j) + 1e-9
    jidx = lax.broadcasted_iota(jnp.int32, (1, _NP), 1).astype(f32)

    keep_ref[...] = jnp.ones((_NP, 1), i8)

    def apply_tile(t, adj, changes):
        rows = pl.ds(t * _R, _R)
        supp = lax.dot_general(
            adj, keep_ref[...],
            (((1,), (0,)), ((), ())),
            preferred_element_type=jnp.int32)
        new32 = jnp.where(supp > 0, 0, 1)
        new = new32.astype(jnp.int8)
        old = keep_ref[rows]
        keep_ref[rows] = new
        return changes + jnp.sum(jnp.abs(new32 - old.astype(jnp.int32)))

    def tile_first(t, changes):
        rows = pl.ds(t * _R, _R)
        off_i = cols_ref[rows, 5:6] * (max_coord + 1.0)
        x0i = cols_ref[rows, 0:1] + off_i
        y0i = cols_ref[rows, 1:2] + off_i
        x1i = cols_ref[rows, 2:3] + off_i
        y1i = cols_ref[rows, 3:4] + off_i
        ai = (x1i - x0i) * (y1i - y0i)
        si = cols_ref[rows, 4:5]
        ii = (lax.broadcasted_iota(jnp.int32, (_R, 1), 0)
              + t * _R).astype(f32)

        wx = jnp.maximum(jnp.minimum(x1i, ox1j) - jnp.maximum(x0i, ox0j), 0.0)
        wy = jnp.maximum(jnp.minimum(y1i, oy1j) - jnp.maximum(y0i, oy0j), 0.0)
        inter = wx * wy
        over = inter * 3.0 > ai + area_jp
        precede = (sj > si) | ((sj == si) & (jidx < ii))
        adj = (over & precede).astype(i8)
        adj_ref[rows, :] = adj
        return apply_tile(t, adj, changes)

    def tile_cached(t, changes):
        return apply_tile(t, adj_ref[pl.ds(t * _R, _R), :], changes)

    def sweep_body(c):
        return lax.fori_loop(0, _NT, tile_cached, 0)

    c1 = lax.fori_loop(0, _NT, tile_first, 0)
    lax.while_loop(lambda c: c > 0, sweep_body, c1)
    keep_out[...] = keep_ref[...].astype(f32)


def _perm(v, idx16):
    return lax.gather(
        v, idx16[:, None],
        lax.GatherDimensionNumbers(
            offset_dims=(), collapsed_slice_dims=(0,), start_index_map=(0,)),
        (1,), mode=lax.GatherScatterMode.PROMISE_IN_BOUNDS)


def _xmax(v):
    lanei = lax.iota(jnp.int32, 16)
    for k in (1, 2, 4, 8):
        v = jnp.maximum(v, _perm(v, lanei ^ k))
    return v


def _xmin(v):
    lanei = lax.iota(jnp.int32, 16)
    for k in (1, 2, 4, 8):
        v = jnp.minimum(v, _perm(v, lanei ^ k))
    return v


def _topk_rounds(read_chunk, nchunks):
    f32 = jnp.float32
    lane = lax.iota(jnp.int32, 16).astype(f32)

    def round_body(r, carry):
        lm, li, resv, resi = carry

        def p1(k, macc):
            v, gi = read_chunk(k)
            elig = (v < lm) | ((v == lm) & (gi > li))
            return jnp.maximum(macc, jnp.where(elig, v, _NEG))

        macc = lax.fori_loop(0, nchunks, p1, jnp.full((16,), _NEG, f32))
        m = _xmax(macc)

        def p2(k, iacc):
            v, gi = read_chunk(k)
            elig = (v < lm) | ((v == lm) & (gi > li))
            hit = elig & (v == m)
            return jnp.minimum(iacc, jnp.where(hit, gi, _BIG))

        iacc = lax.fori_loop(0, nchunks, p2, jnp.full((16,), _BIG, f32))
        idx = _xmin(iacc)
        rf = r.astype(f32)
        resv = jnp.where(lane == rf, m, resv)
        resi = jnp.where(lane == rf, idx, resi)
        return m, idx, resv, resi

    init = (jnp.full((16,), float("inf"), f32), jnp.full((16,), -1.0, f32),
            jnp.full((16,), _NEG, f32), jnp.full((16,), _BIG, f32))
    _, _, resv, resi = lax.fori_loop(0, _MAX_INST, round_body, init)
    return resv, resi


def _select_body(scores_hbm, labels_hbm, keep_hbm, brow_hbm, hid_hbm,
                 hs_out, bx_out,
                 sv, lv, kv, humv, objv, res_stage, shared, merged,
                 idxv, hsv, bxg, sem):
    f32 = jnp.float32
    cid = lax.axis_index("c")
    sid = lax.axis_index("s")
    lane = lax.iota(jnp.int32, 16).astype(f32)

    @pl.when(cid == 0)
    def _local():
        base = sid * _SL
        pltpu.sync_copy(scores_hbm.at[pl.ds(base, _SL)], sv)
        pltpu.sync_copy(labels_hbm.at[pl.ds(base, _SL)], lv)
        pltpu.sync_copy(keep_hbm.at[pl.ds(base, _SL)], kv)

        def mk_masks(k, _):
            sl = pl.ds(k * 16, 16)
            v = sv[sl]
            l = lv[sl]
            kp = kv[sl]
            valid = (kp > 0.5) & (v >= _SCORE_TH)
            humv[sl] = jnp.where(valid & (l == float(_HUMAN_IDX)), v, _NEG)
            objv[sl] = jnp.where(valid & (l != float(_HUMAN_IDX)), v, _NEG)
            return 0

        lax.fori_loop(0, _SLC, mk_masks, 0)
        basef = (sid * _SL).astype(f32)

        def rd_hum(k):
            gi = basef + (k * 16).astype(f32) + lane
            return humv[pl.ds(k * 16, 16)], gi

        def rd_obj(k):
            gi = basef + (k * 16).astype(f32) + lane
            return objv[pl.ds(k * 16, 16)], gi

        hv, hi = _topk_rounds(rd_hum, _SLC)
        ov, oi = _topk_rounds(rd_obj, _SLC)
        res_stage[pl.ds(0, 16)] = hv
        res_stage[pl.ds(16, 16)] = hi
        res_stage[pl.ds(32, 16)] = ov
        res_stage[pl.ds(48, 16)] = oi
        pltpu.sync_copy(res_stage, shared.at[pl.ds(sid * 64, 64)])

    plsc.subcore_barrier()

    @pl.when((cid == 0) & (sid == 0))
    def _merge_and_gather():
        pltpu.sync_copy(shared, merged)

        def rd_cat(off):
            def rd(k):
                return (merged[pl.ds(k * 64 + off, 16)],
                        merged[pl.ds(k * 64 + off + 16, 16)])
            return rd

        ghv, ghi = _topk_rounds(rd_cat(0), _NW)
        gov, goi = _topk_rounds(rd_cat(32), _NW)

        hum_i = jnp.minimum(ghi, _BIG - 1.0).astype(jnp.int32)
        obj_i = jnp.minimum(goi, _BIG - 1.0).astype(jnp.int32)
        hum_i = jnp.where(hum_i >= _NP, 0, hum_i)
        obj_i = jnp.where(obj_i >= _NP, 0, obj_i)
        idxv[pl.ds(0, 16)] = hum_i
        idxv[pl.ds(16, 16)] = obj_i

        pltpu.async_copy(hid_hbm.at[idxv], hsv, sem).wait()
        pltpu.async_copy(brow_hbm.at[idxv], bxg, sem).wait()

        four = jnp.full((16,), 4, jnp.int32)

        def mul_rows(r, _):
            rowh = bxg[r, pl.ds(0, 16)]
            rowo = bxg[r + 16, pl.ds(0, 16)]
            s_h = _perm(rowh, four)
            s_o = _perm(rowo, four)
            sig_h = 1.0 / (1.0 + jnp.exp(-s_h))
            sig_o = 1.0 / (1.0 + jnp.exp(-s_o))

            def mul_chunk(c, _c):
                hsv[r, pl.ds(c * 16, 16)] = hsv[r, pl.ds(c * 16, 16)] * sig_h
                ro = r + 16
                hsv[ro, pl.ds(c * 16, 16)] = hsv[ro, pl.ds(c * 16, 16)] * sig_o
                return 0

            lax.fori_loop(0, 16, mul_chunk, 0)
            return 0

        lax.fori_loop(0, 16, mul_rows, 0)

        pltpu.sync_copy(hsv, hs_out)
        pltpu.sync_copy(bxg, bx_out)


def _make_select():
    mesh = plsc.VectorSubcoreMesh(core_axis_name="c", subcore_axis_name="s")
    f32 = jnp.float32
    return pl.kernel(
        _select_body,
        mesh=mesh,
        out_type=[
            jax.ShapeDtypeStruct((32, 256), f32),
            jax.ShapeDtypeStruct((32, 128), f32),
        ],
        scratch_types=[
            pltpu.VMEM((_SL,), f32),
            pltpu.VMEM((_SL,), f32),
            pltpu.VMEM((_SL,), f32),
            pltpu.VMEM((_SL,), f32),
            pltpu.VMEM((_SL,), f32),
            pltpu.VMEM((64,), f32),
            pltpu.VMEM_SHARED((_NW * 64,), f32),
            pltpu.VMEM((_NW * 64,), f32),
            pltpu.VMEM((32,), jnp.int32),
            pltpu.VMEM((32, 256), f32),
            pltpu.VMEM((32, 128), f32),
            pltpu.SemaphoreType.DMA,
        ],
    )


def kernel(boxes, scores, hidden_states, labels):
    f32 = jnp.float32
    pad = _NP - _N
    boxesp = jnp.pad(boxes, ((0, pad), (0, 0)))
    scoresp = jnp.pad(scores, (0, pad), constant_values=-jnp.inf)
    labelsp = jnp.pad(labels.astype(f32), (0, pad))
    zeros2 = jnp.zeros((_NP, 2), f32)
    cols = jnp.concatenate(
        [boxesp, scoresp[:, None], labelsp[:, None], zeros2], axis=1)
    rows = cols.T

    keep = pl.pallas_call(
        _nms_kernel,
        out_shape=jax.ShapeDtypeStruct((_NP, 1), f32),
        scratch_shapes=[
            pltpu.VMEM((_NP, 1), jnp.int8),
            pltpu.VMEM((_NP, _NP), jnp.int8),
        ],
    )(cols, rows)

    brow = jnp.pad(cols, ((0, 0), (0, 120)))
    hs32, bxg = _make_select()(
        scoresp, labelsp, keep.reshape(_NP), brow, hidden_states)
    hs = jnp.concatenate([hs32[0:_MAX_INST], hs32[16:16 + _MAX_INST]], axis=0)
    bx32 = bxg[:, 0:4]
    bx = jnp.concatenate([bx32[0:_MAX_INST], bx32[16:16 + _MAX_INST]], axis=0)
    return jnp.concatenate([hs, bx], axis=1)


# speedup vs baseline: 1.1048x; 1.0267x over previous
import functools
import jax
import jax.numpy as jnp
from jax import lax
from jax.experimental import pallas as pl
from jax.experimental.pallas import tpu as pltpu
from jax.experimental.pallas import tpu_sc as plsc

_HUMAN_IDX = 0
_SCORE_TH = 0.2
_IOU_TH = 0.5
_MAX_INST = 15
_N = 5000
_NP = 5120
_R = 512
_NT = _NP // _R

_NW = 16
_SL = _NP // _NW
_SLC = _SL // 16
_BIG = 1e9
_NEG = float("-inf")


def _nms_kernel(cols_ref, rows_ref, keep_ref, adj_ref):
    f32 = jnp.float32

    x0j = rows_ref[0:1, :]
    y0j = rows_ref[1:2, :]
    x1j = rows_ref[2:3, :]
    y1j = rows_ref[3:4, :]
    sj = rows_ref[4:5, :]
    labj = rows_ref[5:6, :]

    max_coord = jnp.max(cols_ref[:, 0:4])
    off_j = labj * (max_coord + 1.0)
    ox0j = x0j + off_j
    oy0j = y0j + off_j
    ox1j = x1j + off_j
    oy1j = y1j + off_j
    aj3 = ((ox1j - ox0j) * (oy1j - oy0j) + 1e-9) * (1.0 / 3.0)
    jidx = lax.broadcasted_iota(jnp.int32, (1, _NP), 1).astype(f32)

    keep_ref[...] = jnp.ones((_NP, 1), f32)

    def apply_tile(t, adj, changes):
        rows = pl.ds(t * _R, _R)
        supp = lax.dot_general(
            adj, keep_ref[...],
            (((1,), (0,)), ((), ())),
            preferred_element_type=f32)
        new = jnp.where(supp > 0.5, 0.0, 1.0)
        old = keep_ref[rows]
        keep_ref[rows] = new
        validrow = jnp.where(cols_ref[rows, 4:5] >= _SCORE_TH, 1.0, 0.0)
        return changes + jnp.sum(jnp.abs(new - old) * validrow)

    def tile_first(t, changes):
        rows = pl.ds(t * _R, _R)
        off_i = cols_ref[rows, 5:6] * (max_coord + 1.0)
        x0i = cols_ref[rows, 0:1] + off_i
        y0i = cols_ref[rows, 1:2] + off_i
        x1i = cols_ref[rows, 2:3] + off_i
        y1i = cols_ref[rows, 3:4] + off_i
        ai3 = (x1i - x0i) * (y1i - y0i) * (1.0 / 3.0)
        si = cols_ref[rows, 4:5]
        ii = (lax.broadcasted_iota(jnp.int32, (_R, 1), 0)
              + t * _R).astype(f32)

        wx = jnp.maximum(jnp.minimum(x1i, ox1j) - jnp.maximum(x0i, ox0j), 0.0)
        wy = jnp.maximum(jnp.minimum(y1i, oy1j) - jnp.maximum(y0i, oy0j), 0.0)
        inter = wx * wy
        over = inter > ai3 + aj3
        precede = (sj > si) | ((sj == si) & (jidx < ii))
        adj = (over & precede).astype(f32)
        adj_ref[rows, :] = adj.astype(jnp.int8)
        return apply_tile(t, adj, changes)

    def tile_cached(t, changes):
        adj = adj_ref[pl.ds(t * _R, _R), :].astype(f32)
        return apply_tile(t, adj, changes)

    def sweep_body(c):
        return lax.fori_loop(0, _NT, tile_cached, 0.0)

    c1 = lax.fori_loop(0, _NT, tile_first, 0.0)
    lax.while_loop(lambda c: c > 0.0, sweep_body, c1)


def _perm(v, idx16):
    return lax.gather(
        v, idx16[:, None],
        lax.GatherDimensionNumbers(
            offset_dims=(), collapsed_slice_dims=(0,), start_index_map=(0,)),
        (1,), mode=lax.GatherScatterMode.PROMISE_IN_BOUNDS)


def _xmax(v):
    lanei = lax.iota(jnp.int32, 16)
    for k in (1, 2, 4, 8):
        v = jnp.maximum(v, _perm(v, lanei ^ k))
    return v


def _xmin(v):
    lanei = lax.iota(jnp.int32, 16)
    for k in (1, 2, 4, 8):
        v = jnp.minimum(v, _perm(v, lanei ^ k))
    return v


def _topk_rounds(read_chunk, nchunks):
    f32 = jnp.float32
    lane = lax.iota(jnp.int32, 16).astype(f32)

    def round_body(r, carry):
        lm, li, resv, resi = carry

        def p1(k, macc):
            v, gi = read_chunk(k)
            elig = (v < lm) | ((v == lm) & (gi > li))
            return jnp.maximum(macc, jnp.where(elig, v, _NEG))

        macc = lax.fori_loop(0, nchunks, p1, jnp.full((16,), _NEG, f32))
        m = _xmax(macc)

        def p2(k, iacc):
            v, gi = read_chunk(k)
            elig = (v < lm) | ((v == lm) & (gi > li))
            hit = elig & (v == m)
            return jnp.minimum(iacc, jnp.where(hit, gi, _BIG))

        iacc = lax.fori_loop(0, nchunks, p2, jnp.full((16,), _BIG, f32))
        idx = _xmin(iacc)
        rf = r.astype(f32)
        resv = jnp.where(lane == rf, m, resv)
        resi = jnp.where(lane == rf, idx, resi)
        return m, idx, resv, resi

    init = (jnp.full((16,), float("inf"), f32), jnp.full((16,), -1.0, f32),
            jnp.full((16,), _NEG, f32), jnp.full((16,), _BIG, f32))
    _, _, resv, resi = lax.fori_loop(0, _MAX_INST, round_body, init)
    return resv, resi


def _select_body(scores_hbm, labels_hbm, keep_hbm, brow_hbm, hid_hbm,
                 hs_out, bx_out,
                 sv, lv, kv, humv, objv, res_stage, shared, merged,
                 idxv, hsv, bxg, sem):
    f32 = jnp.float32
    cid = lax.axis_index("c")
    sid = lax.axis_index("s")
    lane = lax.iota(jnp.int32, 16).astype(f32)

    @pl.when(cid == 0)
    def _local():
        base = sid * _SL
        pltpu.sync_copy(scores_hbm.at[pl.ds(base, _SL)], sv)
        pltpu.sync_copy(labels_hbm.at[pl.ds(base, _SL)], lv)
        pltpu.sync_copy(keep_hbm.at[pl.ds(base, _SL)], kv)

        def mk_masks(k, _):
            sl = pl.ds(k * 16, 16)
            v = sv[sl]
            l = lv[sl]
            kp = kv[sl]
            valid = (kp > 0.5) & (v >= _SCORE_TH)
            humv[sl] = jnp.where(valid & (l == float(_HUMAN_IDX)), v, _NEG)
            objv[sl] = jnp.where(valid & (l != float(_HUMAN_IDX)), v, _NEG)
            return 0

        lax.fori_loop(0, _SLC, mk_masks, 0)
        basef = (sid * _SL).astype(f32)

        def rd_hum(k):
            gi = basef + (k * 16).astype(f32) + lane
            return humv[pl.ds(k * 16, 16)], gi

        def rd_obj(k):
            gi = basef + (k * 16).astype(f32) + lane
            return objv[pl.ds(k * 16, 16)], gi

        hv, hi = _topk_rounds(rd_hum, _SLC)
        ov, oi = _topk_rounds(rd_obj, _SLC)
        res_stage[pl.ds(0, 16)] = hv
        res_stage[pl.ds(16, 16)] = hi
        res_stage[pl.ds(32, 16)] = ov
        res_stage[pl.ds(48, 16)] = oi
        pltpu.sync_copy(res_stage, shared.at[pl.ds(sid * 64, 64)])

    plsc.subcore_barrier()

    @pl.when((cid == 0) & (sid == 0))
    def _merge_and_gather():
        pltpu.sync_copy(shared, merged)

        def rd_cat(off):
            def rd(k):
                return (merged[pl.ds(k * 64 + off, 16)],
                        merged[pl.ds(k * 64 + off + 16, 16)])
            return rd

        ghv, ghi = _topk_rounds(rd_cat(0), _NW)
        gov, goi = _topk_rounds(rd_cat(32), _NW)

        hum_i = jnp.minimum(ghi, _BIG - 1.0).astype(jnp.int32)
        obj_i = jnp.minimum(goi, _BIG - 1.0).astype(jnp.int32)
        hum_i = jnp.where(hum_i >= _NP, 0, hum_i)
        obj_i = jnp.where(obj_i >= _NP, 0, obj_i)
        idxv[pl.ds(0, 16)] = hum_i
        idxv[pl.ds(16, 16)] = obj_i

        pltpu.async_copy(hid_hbm.at[idxv], hsv, sem).wait()
        pltpu.async_copy(brow_hbm.at[idxv], bxg, sem).wait()

        four = jnp.full((16,), 4, jnp.int32)

        def mul_rows(r, _):
            rowh = bxg[r, pl.ds(0, 16)]
            rowo = bxg[r + 16, pl.ds(0, 16)]
            s_h = _perm(rowh, four)
            s_o = _perm(rowo, four)
            sig_h = 1.0 / (1.0 + jnp.exp(-s_h))
            sig_o = 1.0 / (1.0 + jnp.exp(-s_o))

            def mul_chunk(c, _c):
                hsv[r, pl.ds(c * 16, 16)] = hsv[r, pl.ds(c * 16, 16)] * sig_h
                ro = r + 16
                hsv[ro, pl.ds(c * 16, 16)] = hsv[ro, pl.ds(c * 16, 16)] * sig_o
                return 0

            lax.fori_loop(0, 16, mul_chunk, 0)
            return 0

        lax.fori_loop(0, 16, mul_rows, 0)

        pltpu.sync_copy(hsv, hs_out)
        pltpu.sync_copy(bxg, bx_out)


def _make_select():
    mesh = plsc.VectorSubcoreMesh(core_axis_name="c", subcore_axis_name="s")
    f32 = jnp.float32
    return pl.kernel(
        _select_body,
        mesh=mesh,
        out_type=[
            jax.ShapeDtypeStruct((32, 256), f32),
            jax.ShapeDtypeStruct((32, 128), f32),
        ],
        scratch_types=[
            pltpu.VMEM((_SL,), f32),
            pltpu.VMEM((_SL,), f32),
            pltpu.VMEM((_SL,), f32),
            pltpu.VMEM((_SL,), f32),
            pltpu.VMEM((_SL,), f32),
            pltpu.VMEM((64,), f32),
            pltpu.VMEM_SHARED((_NW * 64,), f32),
            pltpu.VMEM((_NW * 64,), f32),
            pltpu.VMEM((32,), jnp.int32),
            pltpu.VMEM((32, 256), f32),
            pltpu.VMEM((32, 128), f32),
            pltpu.SemaphoreType.DMA,
        ],
    )


def kernel(boxes, scores, hidden_states, labels):
    f32 = jnp.float32
    pad = _NP - _N
    boxesp = jnp.pad(boxes, ((0, pad), (0, 0)))
    scoresp = jnp.pad(scores, (0, pad), constant_values=-jnp.inf)
    labelsp = jnp.pad(labels.astype(f32), (0, pad))
    zeros2 = jnp.zeros((_NP, 2), f32)
    cols = jnp.concatenate(
        [boxesp, scoresp[:, None], labelsp[:, None], zeros2], axis=1)
    rows = cols.T

    keep = pl.pallas_call(
        _nms_kernel,
        out_shape=jax.ShapeDtypeStruct((_NP, 1), f32),
        scratch_shapes=[
            pltpu.VMEM((_NP, _NP), jnp.int8),
        ],
    )(cols, rows)

    brow = jnp.pad(cols, ((0, 0), (0, 120)))
    hs32, bxg = _make_select()(
        scoresp, labelsp, keep.reshape(_NP), brow, hidden_states)
    hs = jnp.concatenate([hs32[0:_MAX_INST], hs32[16:16 + _MAX_INST]], axis=0)
    bx32 = bxg[:, 0:4]
    bx = jnp.concatenate([bx32[0:_MAX_INST], bx32[16:16 + _MAX_INST]], axis=0)
    return jnp.concatenate([hs, bx], axis=1)
